# Initial kernel scaffold; baseline (speedup 1.0000x reference)
import jax
import jax.numpy as jnp
from jax.experimental import pallas as pl


def kernel(normu, cls):
    raise NotImplementedError("write your pallas kernel here")


# speedup vs baseline: 15.1314x; 15.1314x over previous
import jax
import jax.numpy as jnp
from jax.experimental import pallas as pl

_N = 8192
_D = 1000
_K = 8
_INV_TEMP = 0.5
_BLOCK_ROWS = 256


def _topk_mask_kernel(cls_ref, out_ref):
    x = cls_ref[:]
    m = jnp.max(x, axis=-1, keepdims=True)
    e = jnp.exp((x - m) * _INV_TEMP)
    s = jnp.sum(e, axis=-1, keepdims=True)
    col = jax.lax.broadcasted_iota(jnp.int32, x.shape, 1)
    out = jnp.zeros_like(x)
    for _ in range(_K):
        v = jnp.max(x, axis=-1, keepdims=True)
        idx = jnp.min(jnp.where(x == v, col, _D + 1), axis=-1, keepdims=True)
        onehot = col == idx
        ei = jnp.sum(jnp.where(onehot, e, 0.0), axis=-1, keepdims=True)
        out = out + jnp.where(onehot, ei / s, 0.0)
        s = s - ei
        x = jnp.where(onehot, -jnp.inf, x)
    out_ref[:] = out


def kernel(normu, cls):
    classes = pl.pallas_call(
        _topk_mask_kernel,
        grid=(_N // _BLOCK_ROWS,),
        in_specs=[pl.BlockSpec((_BLOCK_ROWS, _D), lambda i: (i, 0))],
        out_specs=pl.BlockSpec((_BLOCK_ROWS, _D), lambda i: (i, 0)),
        out_shape=jax.ShapeDtypeStruct((_N, _D), jnp.float32),
    )(cls)
    return (normu, classes)


# speedup vs baseline: 16.9584x; 1.1207x over previous
import jax
import jax.numpy as jnp
from jax.experimental import pallas as pl

_N = 8192
_D = 1000
_K = 8
_INV_TEMP = 0.5
_BLOCK_ROWS = 256


def _topk_mask_kernel(cls_ref, out_ref):
    x = cls_ref[:]
    m = jnp.max(x, axis=-1, keepdims=True)
    ew = jnp.exp((x - m) * _INV_TEMP)
    s = jnp.sum(ew, axis=-1, keepdims=True)
    col = jax.lax.broadcasted_iota(jnp.int32, ew.shape, 1)
    out = jnp.zeros_like(ew)
    for _ in range(_K):
        v = jnp.max(ew, axis=-1, keepdims=True)
        idx = jnp.min(jnp.where(ew >= v, col, _D + 1), axis=-1, keepdims=True)
        onehot = col == idx
        out = jnp.where(onehot, v / s, out)
        s = s - v
        ew = jnp.where(onehot, 0.0, ew)
    out_ref[:] = out


def kernel(normu, cls):
    classes = pl.pallas_call(
        _topk_mask_kernel,
        grid=(_N // _BLOCK_ROWS,),
        in_specs=[pl.BlockSpec((_BLOCK_ROWS, _D), lambda i: (i, 0))],
        out_specs=pl.BlockSpec((_BLOCK_ROWS, _D), lambda i: (i, 0)),
        out_shape=jax.ShapeDtypeStruct((_N, _D), jnp.float32),
    )(cls)
    return (normu, classes)


# speedup vs baseline: 20.4991x; 1.2088x over previous
import jax
import jax.numpy as jnp
from jax.experimental import pallas as pl

_N = 8192
_D = 1000
_K = 8
_INV_TEMP = 0.5
_BLOCK_ROWS = 256


def _topk_mask_kernel(cls_ref, out_ref):
    x = cls_ref[:]
    m = jnp.max(x, axis=-1, keepdims=True)
    ew = jnp.exp((x - m) * _INV_TEMP)
    s = jnp.sum(ew, axis=-1, keepdims=True)
    ckey = (
        _D - jax.lax.broadcasted_iota(jnp.int32, ew.shape, 1)
    ).astype(jnp.float32)
    out = jnp.zeros_like(ew)
    for _ in range(_K):
        v = jnp.max(ew, axis=-1, keepdims=True)
        wk = jnp.max(jnp.where(ew >= v, ckey, 0.0), axis=-1, keepdims=True)
        onehot = ckey == wk
        out = jnp.where(onehot, v / s, out)
        s = s - v
        ew = jnp.where(onehot, 0.0, ew)
    out_ref[:] = out


def kernel(normu, cls):
    classes = pl.pallas_call(
        _topk_mask_kernel,
        grid=(_N // _BLOCK_ROWS,),
        in_specs=[pl.BlockSpec((_BLOCK_ROWS, _D), lambda i: (i, 0))],
        out_specs=pl.BlockSpec((_BLOCK_ROWS, _D), lambda i: (i, 0)),
        out_shape=jax.ShapeDtypeStruct((_N, _D), jnp.float32),
    )(cls)
    return (normu, classes)


# speedup vs baseline: 21.2892x; 1.0385x over previous
import jax
import jax.numpy as jnp
from jax.experimental import pallas as pl

_N = 8192
_D = 1000
_K = 8
_INV_TEMP = 0.5
_BLOCK_ROWS = 512


def _topk_mask_kernel(cls_ref, out_ref):
    x = cls_ref[:]
    m = jnp.max(x, axis=-1, keepdims=True)
    ew = jnp.exp((x - m) * _INV_TEMP)
    s = jnp.sum(ew, axis=-1, keepdims=True)
    ckey = (
        _D - jax.lax.broadcasted_iota(jnp.int32, ew.shape, 1)
    ).astype(jnp.float32)
    out = jnp.zeros_like(ew)
    for _ in range(_K):
        v = jnp.max(ew, axis=-1, keepdims=True)
        wk = jnp.max(jnp.where(ew >= v, ckey, 0.0), axis=-1, keepdims=True)
        onehot = ckey == wk
        out = jnp.where(onehot, v / s, out)
        s = s - v
        ew = jnp.where(onehot, 0.0, ew)
    out_ref[:] = out


def kernel(normu, cls):
    classes = pl.pallas_call(
        _topk_mask_kernel,
        grid=(_N // _BLOCK_ROWS,),
        in_specs=[pl.BlockSpec((_BLOCK_ROWS, _D), lambda i: (i, 0))],
        out_specs=pl.BlockSpec((_BLOCK_ROWS, _D), lambda i: (i, 0)),
        out_shape=jax.ShapeDtypeStruct((_N, _D), jnp.float32),
    )(cls)
    return (normu, classes)
